# baseline (speedup 1.0000x reference)
import functools

import jax
import jax.numpy as jnp
from jax import lax
from jax.experimental import pallas as pl
from jax.experimental.pallas import tpu as pltpu
from jax.experimental.pallas import tpu_sc as plsc

_T = 2048
_D = 1024
_F = 1024
_E = 8
_R = 128
_TILE = 128
_G = (_T * 2 + _E * _TILE) // _TILE
_P = _G * _TILE
_STILE = 256
_NC, _NS, _L = 2, 16, 16
_NW = _NC * _NS
_TE = 3 * _L


def _router_body(x_ref, wr_ref, br_ref, wg_ref, bg_ref,
                 ws_ref, wh_ref, e1_ref, e2_ref, w1_ref, w2_ref):
    x = x_ref[...]
    r = jnp.dot(x, wr_ref[...], preferred_element_type=jnp.float32) + br_ref[...]
    logits = jnp.dot(r, wg_ref[...], preferred_element_type=jnp.float32) + bg_ref[...]
    mx = jnp.max(logits, axis=-1, keepdims=True)
    ex = jnp.exp(logits - mx)
    ws = ex / jnp.sum(ex, axis=-1, keepdims=True)
    iota = lax.broadcasted_iota(jnp.int32, (_T, _E), 1)
    m1 = jnp.max(ws, axis=-1, keepdims=True)
    i1 = jnp.min(jnp.where(ws == m1, iota, _E), axis=-1, keepdims=True)
    wsx = jnp.where(iota == i1, -1.0, ws)
    m2 = jnp.max(wsx, axis=-1, keepdims=True)
    i2 = jnp.min(jnp.where(wsx == m2, iota, _E), axis=-1, keepdims=True)
    denom = m1 + m2 + 1e-9
    sel = (iota == i1) | (iota == i2)
    ws_ref[...] = ws
    wh_ref[...] = jnp.where(sel, ws, 0.0) / denom
    e1_ref[...] = i1
    e2_ref[...] = i2
    w1_ref[...] = m1 / denom
    w2_ref[...] = m2 / denom


def _router(x, Wr, br, Wg, bg):
    return pl.pallas_call(
        _router_body,
        out_shape=[
            jax.ShapeDtypeStruct((_T, _E), jnp.float32),
            jax.ShapeDtypeStruct((_T, _E), jnp.float32),
            jax.ShapeDtypeStruct((_T, 1), jnp.int32),
            jax.ShapeDtypeStruct((_T, 1), jnp.int32),
            jax.ShapeDtypeStruct((_T, 1), jnp.float32),
            jax.ShapeDtypeStruct((_T, 1), jnp.float32),
        ],
    )(x, Wr, br, Wg, bg)


def _dispatch_body(e1_hbm, e2_hbm, w1_hbm, w2_hbm,
                   rt_hbm, rs_hbm, p1_hbm, p2_hbm, te_hbm,
                   e1_v, e2_v, w1_v, w2_v, rt_v, rs_v, p1_v, p2_v, te_v):
    wid = lax.axis_index("s") * _NC + lax.axis_index("c")

    @pl.when(wid == 0)
    def _():
        pltpu.sync_copy(e1_hbm, e1_v)
        pltpu.sync_copy(e2_hbm, e2_v)
        pltpu.sync_copy(w1_hbm, w1_v)
        pltpu.sync_copy(w2_hbm, w2_v)
        lanes = lax.iota(jnp.int32, _L)

        def zero_body(i, c):
            rt_v[pl.ds(i * _L, _L)] = (lanes + i * _L) & (_T - 1)
            rs_v[pl.ds(i * _L, _L)] = jnp.zeros((_L,), jnp.float32)
            return c
        lax.fori_loop(0, _P // _L, zero_body, 0)

        def cnt_body(i, cnt):
            sl = pl.ds(i * _L, _L)
            ev1 = e1_v[sl]
            ev2 = e2_v[sl]
            for e in range(_E):
                n = (plsc.all_reduce_population_count(ev1 == e)
                     + plsc.all_reduce_population_count(ev2 == e))
                cnt = jnp.where(lanes == e, cnt + n, cnt)
            return cnt
        cnt = lax.fori_loop(0, _T // _L, cnt_body, jnp.zeros((_L,), jnp.int32))

        padded = ((cnt + (_TILE - 1)) // _TILE) * _TILE
        aoi = plsc.cumsum(padded)
        ao = aoi - padded

        for half in range(_TE // _L):
            gpos = (lanes + _L * half) * _TILE
            tev = jnp.zeros((_L,), jnp.int32)
            for e in range(_E):
                aoe = jnp.sum(jnp.where(lanes == e, ao, 0))
                aie = jnp.sum(jnp.where(lanes == e, aoi, 0))
                m = (gpos >= aoe) & (gpos < aie)
                tev = jnp.where(m, e, tev)
            te_v[pl.ds(half * _L, _L)] = tev

        def pos_body(i, cur):
            sl = pl.ds(i * _L, _L)
            tok = lanes + i * _L
            for (ev, wv, pv) in ((e1_v[sl], w1_v[sl], p1_v),
                                 (e2_v[sl], w2_v[sl], p2_v)):
                for e in range(_E):
                    m = ev == e
                    pc = plsc.cumsum(m.astype(jnp.int32))
                    base = jnp.sum(jnp.where(lanes == e, cur, 0))
                    pos = jnp.maximum(base + pc - 1, 0)
                    plsc.store_scatter(rt_v, [pos], tok, mask=m)
                    plsc.store_scatter(rs_v, [pos], wv, mask=m)
                    plsc.store_scatter(pv, [tok], pos, mask=m)
                    n = plsc.all_reduce_population_count(m)
                    cur = jnp.where(lanes == e, cur + n, cur)
            return cur
        lax.fori_loop(0, _T // _L, pos_body, ao)

        pltpu.sync_copy(rt_v, rt_hbm)
        pltpu.sync_copy(rs_v, rs_hbm)
        pltpu.sync_copy(p1_v, p1_hbm)
        pltpu.sync_copy(p2_v, p2_hbm)
        pltpu.sync_copy(te_v, te_hbm)


def _dispatch(e1, e2, w1n, w2n):
    mesh = plsc.VectorSubcoreMesh(core_axis_name="c", subcore_axis_name="s")
    f = pl.kernel(
        _dispatch_body,
        out_type=[
            jax.ShapeDtypeStruct((_P,), jnp.int32),
            jax.ShapeDtypeStruct((_P,), jnp.float32),
            jax.ShapeDtypeStruct((_T,), jnp.int32),
            jax.ShapeDtypeStruct((_T,), jnp.int32),
            jax.ShapeDtypeStruct((_TE,), jnp.int32),
        ],
        mesh=mesh,
        scratch_types=[
            pltpu.VMEM((_T,), jnp.int32),
            pltpu.VMEM((_T,), jnp.int32),
            pltpu.VMEM((_T,), jnp.float32),
            pltpu.VMEM((_T,), jnp.float32),
            pltpu.VMEM((_P,), jnp.int32),
            pltpu.VMEM((_P,), jnp.float32),
            pltpu.VMEM((_T,), jnp.int32),
            pltpu.VMEM((_T,), jnp.int32),
            pltpu.VMEM((_TE,), jnp.int32),
        ],
        compiler_params=pltpu.CompilerParams(needs_layout_passes=False),
    )
    return f(e1, e2, w1n, w2n)


_GCHUNK = 40
_GN = (_P // _NW) // _GCHUNK


def _gather_body(x_hbm, rt_hbm, xs_hbm, idx_v, rows0, rows1, gsem, wsem):
    wid = lax.axis_index("s") * _NC + lax.axis_index("c")
    rows_per_w = _P // _NW
    base = wid * rows_per_w
    pltpu.sync_copy(rt_hbm.at[pl.ds(base, rows_per_w)], idx_v)
    bufs = (rows0, rows1)
    gathers = [pltpu.async_copy(
        x_hbm.at[idx_v.at[pl.ds(c * _GCHUNK, _GCHUNK)]], bufs[c % 2], gsem)
        for c in range(1)]
    writes = []
    for c in range(_GN):
        if c + 1 < _GN:
            if c >= 1:
                writes[c - 1].wait()
            gathers.append(pltpu.async_copy(
                x_hbm.at[idx_v.at[pl.ds((c + 1) * _GCHUNK, _GCHUNK)]],
                bufs[(c + 1) % 2], gsem))
        gathers[c].wait()
        writes.append(pltpu.async_copy(
            bufs[c % 2], xs_hbm.at[pl.ds(base + c * _GCHUNK, _GCHUNK)], wsem))
    writes[-2].wait()
    writes[-1].wait()


def _gather(x, rt):
    mesh = plsc.VectorSubcoreMesh(core_axis_name="c", subcore_axis_name="s")
    f = pl.kernel(
        _gather_body,
        out_type=jax.ShapeDtypeStruct((_P, _D), jnp.float32),
        mesh=mesh,
        scratch_types=[
            pltpu.VMEM((_P // _NW,), jnp.int32),
            pltpu.VMEM((_GCHUNK, _D), jnp.float32),
            pltpu.VMEM((_GCHUNK, _D), jnp.float32),
            pltpu.SemaphoreType.DMA,
            pltpu.SemaphoreType.DMA,
        ],
    )
    return f(x, rt)


def _ffn_body(te_ref, xs_ref, w1_ref, b1_ref, w2_ref, b2_ref, rs_ref, ys_ref):
    xs = xs_ref[...].astype(jnp.bfloat16)
    h = jnp.dot(xs, w1_ref[0], preferred_element_type=jnp.float32) + b1_ref[0]
    h = jax.nn.gelu(h)
    y = jnp.dot(h.astype(jnp.bfloat16), w2_ref[0],
                preferred_element_type=jnp.float32) + b2_ref[0]
    ys_ref[...] = y * rs_ref[0]


def _ffn(te, xs, W1, b1, W2, b2, rs):
    grid_spec = pltpu.PrefetchScalarGridSpec(
        num_scalar_prefetch=1,
        grid=(_G,),
        in_specs=[
            pl.BlockSpec((_TILE, _D), lambda g, te: (g, 0)),
            pl.BlockSpec((1, _D, _F), lambda g, te: (te[g], 0, 0)),
            pl.BlockSpec((1, 1, _F), lambda g, te: (te[g], 0, 0)),
            pl.BlockSpec((1, _F, _D), lambda g, te: (te[g], 0, 0)),
            pl.BlockSpec((1, 1, _D), lambda g, te: (te[g], 0, 0)),
            pl.BlockSpec((1, _TILE, 1), lambda g, te: (g, 0, 0)),
        ],
        out_specs=pl.BlockSpec((_TILE, _D), lambda g, te: (g, 0)),
    )
    return pl.pallas_call(
        _ffn_body,
        grid_spec=grid_spec,
        out_shape=jax.ShapeDtypeStruct((_P, _D), jnp.float32),
        compiler_params=pltpu.CompilerParams(
            dimension_semantics=("arbitrary",)),
    )(te, xs, W1, b1, W2, b2, rs)


def _shared_body(x_ref, w1_ref, b1_ref, w2_ref, b2_ref, o_ref):
    h = jnp.dot(x_ref[...], w1_ref[...], preferred_element_type=jnp.float32)
    h = jax.nn.gelu(h + b1_ref[...])
    o_ref[...] = jnp.dot(h, w2_ref[...],
                         preferred_element_type=jnp.float32) + b2_ref[...]


def _shared(x, Ws1, bs1, Ws2, bs2):
    return pl.pallas_call(
        _shared_body,
        grid=(_T // _STILE,),
        in_specs=[
            pl.BlockSpec((_STILE, _D), lambda g: (g, 0)),
            pl.BlockSpec((_D, _F), lambda g: (0, 0)),
            pl.BlockSpec((1, _F), lambda g: (0, 0)),
            pl.BlockSpec((_F, _D), lambda g: (0, 0)),
            pl.BlockSpec((1, _D), lambda g: (0, 0)),
        ],
        out_specs=pl.BlockSpec((_STILE, _D), lambda g: (g, 0)),
        out_shape=jax.ShapeDtypeStruct((_T, _D), jnp.float32),
        compiler_params=pltpu.CompilerParams(
            dimension_semantics=("arbitrary",)),
    )(x, Ws1, bs1, Ws2, bs2)


_CCHUNK = 16
_CN = (_T // _NW) // _CCHUNK


def _combine_body(ys_hbm, sh_hbm, p1_hbm, p2_hbm, out_hbm,
                  acc0, acc1, ra0, ra1, rb0, rb1, i1_v, i2_v,
                  ssem, gsem, wsem):
    wid = lax.axis_index("s") * _NC + lax.axis_index("c")
    toks_per_w = _T // _NW
    base = wid * toks_per_w
    pltpu.sync_copy(p1_hbm.at[pl.ds(base, toks_per_w)], i1_v)
    pltpu.sync_copy(p2_hbm.at[pl.ds(base, toks_per_w)], i2_v)
    accs = (acc0, acc1)
    ras = (ra0, ra1)
    rbs = (rb0, rb1)

    def start(c):
        off = base + c * _CCHUNK
        sl = pl.ds(c * _CCHUNK, _CCHUNK)
        return (pltpu.async_copy(sh_hbm.at[pl.ds(off, _CCHUNK)],
                                 accs[c % 2], ssem),
                pltpu.async_copy(ys_hbm.at[i1_v.at[sl]], ras[c % 2], gsem),
                pltpu.async_copy(ys_hbm.at[i2_v.at[sl]], rbs[c % 2], gsem))

    pend = start(0)
    writes = []
    for c in range(_CN):
        if c + 1 < _CN:
            if c >= 1:
                writes[c - 1].wait()
            nxt = start(c + 1)
        for d in pend:
            d.wait()
        acc, ra, rb = accs[c % 2], ras[c % 2], rbs[c % 2]
        for r in range(_CCHUNK):
            @plsc.parallel_loop(0, _D // _L, unroll=8)
            def _(cc):
                sl = pl.ds(cc * _L, _L)
                plsc.addupdate(acc.at[r, sl], ra[r, sl] + rb[r, sl])
        writes.append(pltpu.async_copy(
            acc, out_hbm.at[pl.ds(base + c * _CCHUNK, _CCHUNK)], wsem))
        if c + 1 < _CN:
            pend = nxt
    writes[-2].wait()
    writes[-1].wait()


def _combine(ys, sh, p1, p2):
    mesh = plsc.VectorSubcoreMesh(core_axis_name="c", subcore_axis_name="s")
    f = pl.kernel(
        _combine_body,
        out_type=jax.ShapeDtypeStruct((_T, _D), jnp.float32),
        mesh=mesh,
        scratch_types=[
            pltpu.VMEM((_CCHUNK, _D), jnp.float32),
            pltpu.VMEM((_CCHUNK, _D), jnp.float32),
            pltpu.VMEM((_CCHUNK, _D), jnp.float32),
            pltpu.VMEM((_CCHUNK, _D), jnp.float32),
            pltpu.VMEM((_CCHUNK, _D), jnp.float32),
            pltpu.VMEM((_CCHUNK, _D), jnp.float32),
            pltpu.VMEM((_T // _NW,), jnp.int32),
            pltpu.VMEM((_T // _NW,), jnp.int32),
            pltpu.SemaphoreType.DMA,
            pltpu.SemaphoreType.DMA,
            pltpu.SemaphoreType.DMA,
        ],
        compiler_params=pltpu.CompilerParams(needs_layout_passes=False),
    )
    return f(ys, sh, p1, p2)


def kernel(x, Wr, br, Wg, bg, W1, b1, W2, b2, Ws1, bs1, Ws2, bs2):
    ws, wh, e1, e2, w1n, w2n = _router(
        x, Wr, br.reshape(1, _R), Wg, bg.reshape(1, _E))
    sh = _shared(x, Ws1, bs1.reshape(1, _F), Ws2, bs2.reshape(1, _D))
    rt, rs, p1, p2, te = _dispatch(
        e1.reshape(_T), e2.reshape(_T), w1n.reshape(_T), w2n.reshape(_T))
    xs = _gather(x, rt)
    ys = _ffn(te, xs, W1.astype(jnp.bfloat16), b1.reshape(_E, 1, _F),
              W2.astype(jnp.bfloat16), b2.reshape(_E, 1, _D),
              rs.reshape(_G, _TILE, 1))
    out = _combine(ys, sh, p1, p2)
    return out, wh, ws


# speedup vs baseline: 1.1700x; 1.1700x over previous
import functools

import jax
import jax.numpy as jnp
from jax import lax
from jax.experimental import pallas as pl
from jax.experimental.pallas import tpu as pltpu
from jax.experimental.pallas import tpu_sc as plsc

_T = 2048
_D = 1024
_F = 1024
_E = 8
_R = 128
_TILE = 128
_G = (_T * 2 + _E * _TILE) // _TILE
_P = _G * _TILE
_STILE = 256
_NC, _NS, _L = 2, 16, 16
_NW = _NC * _NS
_TE = 3 * _L


def _router_body(x_ref, wr_ref, br_ref, wg_ref, bg_ref,
                 ws_ref, wh_ref, e1_ref, e2_ref, w1_ref, w2_ref):
    x = x_ref[...]
    r = jnp.dot(x, wr_ref[...], preferred_element_type=jnp.float32) + br_ref[...]
    logits = jnp.dot(r, wg_ref[...], preferred_element_type=jnp.float32) + bg_ref[...]
    mx = jnp.max(logits, axis=-1, keepdims=True)
    ex = jnp.exp(logits - mx)
    ws = ex / jnp.sum(ex, axis=-1, keepdims=True)
    iota = lax.broadcasted_iota(jnp.int32, (_T, _E), 1)
    m1 = jnp.max(ws, axis=-1, keepdims=True)
    i1 = jnp.min(jnp.where(ws == m1, iota, _E), axis=-1, keepdims=True)
    wsx = jnp.where(iota == i1, -1.0, ws)
    m2 = jnp.max(wsx, axis=-1, keepdims=True)
    i2 = jnp.min(jnp.where(wsx == m2, iota, _E), axis=-1, keepdims=True)
    denom = m1 + m2 + 1e-9
    sel = (iota == i1) | (iota == i2)
    ws_ref[...] = ws
    wh_ref[...] = jnp.where(sel, ws, 0.0) / denom
    e1_ref[...] = i1
    e2_ref[...] = i2
    w1_ref[...] = m1 / denom
    w2_ref[...] = m2 / denom


def _router(x, Wr, br, Wg, bg):
    return pl.pallas_call(
        _router_body,
        out_shape=[
            jax.ShapeDtypeStruct((_T, _E), jnp.float32),
            jax.ShapeDtypeStruct((_T, _E), jnp.float32),
            jax.ShapeDtypeStruct((_T, 1), jnp.int32),
            jax.ShapeDtypeStruct((_T, 1), jnp.int32),
            jax.ShapeDtypeStruct((_T, 1), jnp.float32),
            jax.ShapeDtypeStruct((_T, 1), jnp.float32),
        ],
    )(x, Wr, br, Wg, bg)


def _dispatch_body(e1_hbm, e2_hbm, w1_hbm, w2_hbm,
                   rt_hbm, rs_hbm, p1_hbm, p2_hbm, te_hbm,
                   e1_v, e2_v, w1_v, w2_v, rt_v, rs_v, p1_v, p2_v, te_v):
    wid = lax.axis_index("s") * _NC + lax.axis_index("c")

    @pl.when(wid == 0)
    def _():
        pltpu.sync_copy(e1_hbm, e1_v)
        pltpu.sync_copy(e2_hbm, e2_v)
        pltpu.sync_copy(w1_hbm, w1_v)
        pltpu.sync_copy(w2_hbm, w2_v)
        lanes = lax.iota(jnp.int32, _L)

        def zero_body(i, c):
            rt_v[pl.ds(i * _L, _L)] = (lanes + i * _L) & (_T - 1)
            rs_v[pl.ds(i * _L, _L)] = jnp.zeros((_L,), jnp.float32)
            return c
        lax.fori_loop(0, _P // _L, zero_body, 0)

        def cnt_body(i, cnt):
            sl = pl.ds(i * _L, _L)
            ev1 = e1_v[sl]
            ev2 = e2_v[sl]
            for e in range(_E):
                n = (plsc.all_reduce_population_count(ev1 == e)
                     + plsc.all_reduce_population_count(ev2 == e))
                cnt = jnp.where(lanes == e, cnt + n, cnt)
            return cnt
        cnt = lax.fori_loop(0, _T // _L, cnt_body, jnp.zeros((_L,), jnp.int32))

        padded = ((cnt + (_TILE - 1)) // _TILE) * _TILE
        aoi = plsc.cumsum(padded)
        ao = aoi - padded

        for half in range(_TE // _L):
            gpos = (lanes + _L * half) * _TILE
            tev = jnp.zeros((_L,), jnp.int32)
            for e in range(_E):
                aoe = jnp.sum(jnp.where(lanes == e, ao, 0))
                aie = jnp.sum(jnp.where(lanes == e, aoi, 0))
                m = (gpos >= aoe) & (gpos < aie)
                tev = jnp.where(m, e, tev)
            te_v[pl.ds(half * _L, _L)] = tev

        def pos_body(i, cur):
            sl = pl.ds(i * _L, _L)
            tok = lanes + i * _L
            for (ev, wv, pv) in ((e1_v[sl], w1_v[sl], p1_v),
                                 (e2_v[sl], w2_v[sl], p2_v)):
                for e in range(_E):
                    m = ev == e
                    pc = plsc.cumsum(m.astype(jnp.int32))
                    base = jnp.sum(jnp.where(lanes == e, cur, 0))
                    pos = jnp.maximum(base + pc - 1, 0)
                    plsc.store_scatter(rt_v, [pos], tok, mask=m)
                    plsc.store_scatter(rs_v, [pos], wv, mask=m)
                    plsc.store_scatter(pv, [tok], pos, mask=m)
                    n = plsc.all_reduce_population_count(m)
                    cur = jnp.where(lanes == e, cur + n, cur)
            return cur
        lax.fori_loop(0, _T // _L, pos_body, ao)

        pltpu.sync_copy(rt_v, rt_hbm)
        pltpu.sync_copy(rs_v, rs_hbm)
        pltpu.sync_copy(p1_v, p1_hbm)
        pltpu.sync_copy(p2_v, p2_hbm)
        pltpu.sync_copy(te_v, te_hbm)


def _dispatch(e1, e2, w1n, w2n):
    mesh = plsc.VectorSubcoreMesh(core_axis_name="c", subcore_axis_name="s")
    f = pl.kernel(
        _dispatch_body,
        out_type=[
            jax.ShapeDtypeStruct((_P,), jnp.int32),
            jax.ShapeDtypeStruct((_P,), jnp.float32),
            jax.ShapeDtypeStruct((_T,), jnp.int32),
            jax.ShapeDtypeStruct((_T,), jnp.int32),
            jax.ShapeDtypeStruct((_TE,), jnp.int32),
        ],
        mesh=mesh,
        scratch_types=[
            pltpu.VMEM((_T,), jnp.int32),
            pltpu.VMEM((_T,), jnp.int32),
            pltpu.VMEM((_T,), jnp.float32),
            pltpu.VMEM((_T,), jnp.float32),
            pltpu.VMEM((_P,), jnp.int32),
            pltpu.VMEM((_P,), jnp.float32),
            pltpu.VMEM((_T,), jnp.int32),
            pltpu.VMEM((_T,), jnp.int32),
            pltpu.VMEM((_TE,), jnp.int32),
        ],
        compiler_params=pltpu.CompilerParams(needs_layout_passes=False),
    )
    return f(e1, e2, w1n, w2n)


_GCHUNK = 40
_GN = (_P // _NW) // _GCHUNK


def _gather_body(x_hbm, rt_hbm, xs_hbm, idx_v, rows0, rows1, gsem, wsem):
    wid = lax.axis_index("s") * _NC + lax.axis_index("c")
    rows_per_w = _P // _NW
    base = wid * rows_per_w
    pltpu.sync_copy(rt_hbm.at[pl.ds(base, rows_per_w)], idx_v)
    bufs = (rows0, rows1)
    gathers = [pltpu.async_copy(
        x_hbm.at[idx_v.at[pl.ds(c * _GCHUNK, _GCHUNK)]], bufs[c % 2], gsem)
        for c in range(1)]
    writes = []
    for c in range(_GN):
        if c + 1 < _GN:
            if c >= 1:
                writes[c - 1].wait()
            gathers.append(pltpu.async_copy(
                x_hbm.at[idx_v.at[pl.ds((c + 1) * _GCHUNK, _GCHUNK)]],
                bufs[(c + 1) % 2], gsem))
        gathers[c].wait()
        writes.append(pltpu.async_copy(
            bufs[c % 2], xs_hbm.at[pl.ds(base + c * _GCHUNK, _GCHUNK)], wsem))
    writes[-2].wait()
    writes[-1].wait()


def _gather(x, rt):
    mesh = plsc.VectorSubcoreMesh(core_axis_name="c", subcore_axis_name="s")
    f = pl.kernel(
        _gather_body,
        out_type=jax.ShapeDtypeStruct((_P, _D), jnp.float32),
        mesh=mesh,
        scratch_types=[
            pltpu.VMEM((_P // _NW,), jnp.int32),
            pltpu.VMEM((_GCHUNK, _D), jnp.float32),
            pltpu.VMEM((_GCHUNK, _D), jnp.float32),
            pltpu.SemaphoreType.DMA,
            pltpu.SemaphoreType.DMA,
        ],
    )
    return f(x, rt)


def _ffn_body(te_ref, xs_ref, w1_ref, b1_ref, w2_ref, b2_ref, rs_ref, ys_ref):
    xs = xs_ref[...]
    h = jnp.dot(xs, w1_ref[0], preferred_element_type=jnp.float32) + b1_ref[0]
    h = jax.nn.gelu(h)
    y = jnp.dot(h, w2_ref[0], preferred_element_type=jnp.float32) + b2_ref[0]
    ys_ref[...] = y * rs_ref[0]


def _ffn(te, xs, W1, b1, W2, b2, rs):
    grid_spec = pltpu.PrefetchScalarGridSpec(
        num_scalar_prefetch=1,
        grid=(_G,),
        in_specs=[
            pl.BlockSpec((_TILE, _D), lambda g, te: (g, 0)),
            pl.BlockSpec((1, _D, _F), lambda g, te: (te[g], 0, 0)),
            pl.BlockSpec((1, 1, _F), lambda g, te: (te[g], 0, 0)),
            pl.BlockSpec((1, _F, _D), lambda g, te: (te[g], 0, 0)),
            pl.BlockSpec((1, 1, _D), lambda g, te: (te[g], 0, 0)),
            pl.BlockSpec((1, _TILE, 1), lambda g, te: (g, 0, 0)),
        ],
        out_specs=pl.BlockSpec((_TILE, _D), lambda g, te: (g, 0)),
    )
    return pl.pallas_call(
        _ffn_body,
        grid_spec=grid_spec,
        out_shape=jax.ShapeDtypeStruct((_P, _D), jnp.float32),
        compiler_params=pltpu.CompilerParams(
            dimension_semantics=("arbitrary",)),
    )(te, xs, W1, b1, W2, b2, rs)


def _shared_body(x_ref, w1_ref, b1_ref, w2_ref, b2_ref, o_ref):
    h = jnp.dot(x_ref[...], w1_ref[...], preferred_element_type=jnp.float32)
    h = jax.nn.gelu(h + b1_ref[...])
    o_ref[...] = jnp.dot(h, w2_ref[...],
                         preferred_element_type=jnp.float32) + b2_ref[...]


def _shared(x, Ws1, bs1, Ws2, bs2):
    return pl.pallas_call(
        _shared_body,
        grid=(_T // _STILE,),
        in_specs=[
            pl.BlockSpec((_STILE, _D), lambda g: (g, 0)),
            pl.BlockSpec((_D, _F), lambda g: (0, 0)),
            pl.BlockSpec((1, _F), lambda g: (0, 0)),
            pl.BlockSpec((_F, _D), lambda g: (0, 0)),
            pl.BlockSpec((1, _D), lambda g: (0, 0)),
        ],
        out_specs=pl.BlockSpec((_STILE, _D), lambda g: (g, 0)),
        out_shape=jax.ShapeDtypeStruct((_T, _D), jnp.float32),
        compiler_params=pltpu.CompilerParams(
            dimension_semantics=("arbitrary",)),
    )(x, Ws1, bs1, Ws2, bs2)


_CCHUNK = 16
_CN = (_T // _NW) // _CCHUNK


def _combine_body(ys_hbm, sh_hbm, p1_hbm, p2_hbm, out_hbm,
                  acc0, acc1, ra0, ra1, rb0, rb1, i1_v, i2_v,
                  ssem, gsem, wsem):
    wid = lax.axis_index("s") * _NC + lax.axis_index("c")
    toks_per_w = _T // _NW
    base = wid * toks_per_w
    pltpu.sync_copy(p1_hbm.at[pl.ds(base, toks_per_w)], i1_v)
    pltpu.sync_copy(p2_hbm.at[pl.ds(base, toks_per_w)], i2_v)
    accs = (acc0, acc1)
    ras = (ra0, ra1)
    rbs = (rb0, rb1)

    def start(c):
        off = base + c * _CCHUNK
        sl = pl.ds(c * _CCHUNK, _CCHUNK)
        return (pltpu.async_copy(sh_hbm.at[pl.ds(off, _CCHUNK)],
                                 accs[c % 2], ssem),
                pltpu.async_copy(ys_hbm.at[i1_v.at[sl]], ras[c % 2], gsem),
                pltpu.async_copy(ys_hbm.at[i2_v.at[sl]], rbs[c % 2], gsem))

    pend = start(0)
    writes = []
    for c in range(_CN):
        if c + 1 < _CN:
            if c >= 1:
                writes[c - 1].wait()
            nxt = start(c + 1)
        for d in pend:
            d.wait()
        acc, ra, rb = accs[c % 2], ras[c % 2], rbs[c % 2]
        for r in range(_CCHUNK):
            @plsc.parallel_loop(0, _D // _L, unroll=8)
            def _(cc):
                sl = pl.ds(cc * _L, _L)
                plsc.addupdate(acc.at[r, sl], ra[r, sl] + rb[r, sl])
        writes.append(pltpu.async_copy(
            acc, out_hbm.at[pl.ds(base + c * _CCHUNK, _CCHUNK)], wsem))
        if c + 1 < _CN:
            pend = nxt
    writes[-2].wait()
    writes[-1].wait()


def _combine(ys, sh, p1, p2):
    mesh = plsc.VectorSubcoreMesh(core_axis_name="c", subcore_axis_name="s")
    f = pl.kernel(
        _combine_body,
        out_type=jax.ShapeDtypeStruct((_T, _D), jnp.float32),
        mesh=mesh,
        scratch_types=[
            pltpu.VMEM((_CCHUNK, _D), jnp.float32),
            pltpu.VMEM((_CCHUNK, _D), jnp.float32),
            pltpu.VMEM((_CCHUNK, _D), jnp.float32),
            pltpu.VMEM((_CCHUNK, _D), jnp.float32),
            pltpu.VMEM((_CCHUNK, _D), jnp.float32),
            pltpu.VMEM((_CCHUNK, _D), jnp.float32),
            pltpu.VMEM((_T // _NW,), jnp.int32),
            pltpu.VMEM((_T // _NW,), jnp.int32),
            pltpu.SemaphoreType.DMA,
            pltpu.SemaphoreType.DMA,
            pltpu.SemaphoreType.DMA,
        ],
        compiler_params=pltpu.CompilerParams(needs_layout_passes=False),
    )
    return f(ys, sh, p1, p2)


def kernel(x, Wr, br, Wg, bg, W1, b1, W2, b2, Ws1, bs1, Ws2, bs2):
    ws, wh, e1, e2, w1n, w2n = _router(
        x, Wr, br.reshape(1, _R), Wg, bg.reshape(1, _E))
    sh = _shared(x, Ws1, bs1.reshape(1, _F), Ws2, bs2.reshape(1, _D))
    rt, rs, p1, p2, te = _dispatch(
        e1.reshape(_T), e2.reshape(_T), w1n.reshape(_T), w2n.reshape(_T))
    xs = _gather(x, rt)
    ys = _ffn(te, xs, W1, b1.reshape(_E, 1, _F), W2, b2.reshape(_E, 1, _D),
              rs.reshape(_G, _TILE, 1))
    out = _combine(ys, sh, p1, p2)
    return out, wh, ws
